# baseline (speedup 1.0000x reference)
import jax
import jax.numpy as jnp
from jax.experimental import pallas as pl
from jax.experimental.pallas import tpu as pltpu

B = 64
D = 1024
MAX_SEQ = 256
L_TXT = MAX_SEQ - 4

SB = 32
NS = MAX_SEQ // SB


def _body(t_ref, txt_ref, ctxt_ref, img_ref, pe_ref, fin_ref, g_ref, b_ref,
          out_ref, bot_ref):
    s = pl.program_id(0)

    @pl.when(s == 0)
    def _():
        k = jax.lax.broadcasted_iota(jnp.int32, (B, D), 1)
        idx = jnp.where(k < D // 2, k, k - D // 2).astype(jnp.float32)
        inv_freq = jnp.exp(idx * (-jnp.log(10000.0) / (D // 2)))
        phase = jnp.where(k < D // 2, 0.0, jnp.pi / 2)
        row = jax.lax.broadcasted_iota(jnp.int32, (B, 1), 0)
        tvec = jnp.zeros((B, 1), jnp.float32)
        for i in range(B):
            tvec = jnp.where(row == i, t_ref[i].astype(jnp.float32), tvec)
        temb = jnp.sin(tvec * inv_freq + phase)
        bot_ref[...] = jnp.stack(
            [ctxt_ref[...], temb, img_ref[...],
             jnp.broadcast_to(fin_ref[...], (B, D))], axis=0)

    txt = txt_ref[...]
    pe = pe_ref[...][:, None, :]

    gmask = (jax.lax.broadcasted_iota(jnp.int32, (SB, 1, 1), 0) + s * SB
             < L_TXT)
    tail = jnp.concatenate(
        [jnp.zeros((SB - 4, B, D), jnp.float32), bot_ref[...]], axis=0)
    x = jnp.where(gmask, txt, tail) + pe

    s1 = jnp.sum(x, axis=2, keepdims=True)
    s2 = jnp.sum(x * x, axis=2, keepdims=True)
    mean = s1 * (1.0 / D)
    var = s2 * (1.0 / D) - mean * mean
    r = jax.lax.rsqrt(var + 1e-5)
    y = (x - mean) * r * g_ref[...][None] + b_ref[...][None]
    out_ref[...] = jnp.transpose(y, (1, 0, 2))


@jax.jit
def kernel(clip_img_emb, t, encoded_txt, clip_txt_emb, pos_emb, final_emb,
           ln_gamma, ln_beta):
    grid = (NS,)
    out = pl.pallas_call(
        _body,
        grid=grid,
        in_specs=[
            pl.BlockSpec(memory_space=pltpu.SMEM),
            pl.BlockSpec((SB, B, D), lambda s: (s, 0, 0)),
            pl.BlockSpec((B, D), lambda s: (0, 0)),
            pl.BlockSpec((B, D), lambda s: (0, 0)),
            pl.BlockSpec((SB, D), lambda s: (s, 0)),
            pl.BlockSpec((1, D), lambda s: (0, 0)),
            pl.BlockSpec((1, D), lambda s: (0, 0)),
            pl.BlockSpec((1, D), lambda s: (0, 0)),
        ],
        out_specs=pl.BlockSpec((B, SB, D), lambda s: (0, s, 0)),
        out_shape=jax.ShapeDtypeStruct((B, MAX_SEQ, D), jnp.float32),
        scratch_shapes=[pltpu.VMEM((4, B, D), jnp.float32)],
        compiler_params=pltpu.CompilerParams(
            dimension_semantics=("arbitrary",)),
    )(t, encoded_txt.transpose(1, 0, 2), clip_txt_emb,
      clip_img_emb, pos_emb, final_emb[None, :], ln_gamma[None, :],
      ln_beta[None, :])
    return out


# speedup vs baseline: 1.0073x; 1.0073x over previous
import jax
import jax.numpy as jnp
from jax.experimental import pallas as pl
from jax.experimental.pallas import tpu as pltpu

B = 64
D = 1024
MAX_SEQ = 256
L_TXT = MAX_SEQ - 4

NB = 8


def _body(t_ref, txt_ref, ctxt_ref, img_ref, pe_ref, fin_ref, g_ref, b_ref,
          out_ref, peb_ref):
    bb = pl.program_id(0)

    @pl.when(bb == 0)
    def _():
        peb_ref[...] = jnp.broadcast_to(pe_ref[...][:, None, :],
                                        (MAX_SEQ, NB, D))
    txt = txt_ref[...]

    k = jax.lax.broadcasted_iota(jnp.int32, (NB, D), 1)
    idx = jnp.where(k < D // 2, k, k - D // 2).astype(jnp.float32)
    inv_freq = jnp.exp(idx * (-jnp.log(10000.0) / (D // 2)))
    phase = jnp.where(k < D // 2, 0.0, jnp.pi / 2)
    row = jax.lax.broadcasted_iota(jnp.int32, (NB, 1), 0)
    tvec = jnp.zeros((NB, 1), jnp.float32)
    for i in range(NB):
        tvec = jnp.where(row == i, t_ref[bb * NB + i].astype(jnp.float32),
                         tvec)
    temb = jnp.sin(tvec * inv_freq + phase)

    bot = jnp.stack(
        [ctxt_ref[...], temb, img_ref[...],
         jnp.broadcast_to(fin_ref[...], (NB, D))], axis=0)

    x = jnp.concatenate([txt, bot], axis=0) + peb_ref[...]

    s1 = jnp.sum(x, axis=2, keepdims=True)
    s2 = jnp.sum(x * x, axis=2, keepdims=True)
    mean = s1 * (1.0 / D)
    var = s2 * (1.0 / D) - mean * mean
    r = jax.lax.rsqrt(var + 1e-5)
    y = (x - mean) * r * g_ref[...][None] + b_ref[...][None]
    out_ref[...] = jnp.transpose(y, (1, 0, 2))


@jax.jit
def kernel(clip_img_emb, t, encoded_txt, clip_txt_emb, pos_emb, final_emb,
           ln_gamma, ln_beta):
    grid = (B // NB,)
    out = pl.pallas_call(
        _body,
        grid=grid,
        in_specs=[
            pl.BlockSpec(memory_space=pltpu.SMEM),
            pl.BlockSpec((L_TXT, NB, D), lambda b: (0, b, 0)),
            pl.BlockSpec((NB, D), lambda b: (b, 0)),
            pl.BlockSpec((NB, D), lambda b: (b, 0)),
            pl.BlockSpec((MAX_SEQ, D), lambda b: (0, 0)),
            pl.BlockSpec((1, D), lambda b: (0, 0)),
            pl.BlockSpec((1, D), lambda b: (0, 0)),
            pl.BlockSpec((1, D), lambda b: (0, 0)),
        ],
        out_specs=pl.BlockSpec((NB, MAX_SEQ, D), lambda b: (b, 0, 0)),
        out_shape=jax.ShapeDtypeStruct((B, MAX_SEQ, D), jnp.float32),
        scratch_shapes=[pltpu.VMEM((MAX_SEQ, NB, D), jnp.float32)],
        compiler_params=pltpu.CompilerParams(
            dimension_semantics=("arbitrary",)),
    )(t, encoded_txt.transpose(1, 0, 2), clip_txt_emb,
      clip_img_emb, pos_emb, final_emb[None, :], ln_gamma[None, :],
      ln_beta[None, :])
    return out
